# Initial kernel scaffold; baseline (speedup 1.0000x reference)
import jax
import jax.numpy as jnp
from jax.experimental import pallas as pl


def kernel(queries, keys, values):
    raise NotImplementedError("write your pallas kernel here")


# speedup vs baseline: 1.8986x; 1.8986x over previous
import functools
import math

import jax
import jax.numpy as jnp
import numpy as np
from jax import lax
from jax.experimental import pallas as pl
from jax.experimental.pallas import tpu as pltpu

_NEG = -3.0e38


@functools.lru_cache(maxsize=4)
def _sample_count_matrix(Lq: int, Lk: int, sample_k: int) -> np.ndarray:
    cpu = jax.devices("cpu")[0]
    with jax.ensure_compile_time_eval(), jax.default_device(cpu):
        idx = jax.random.randint(jax.random.key(42), (Lq, sample_k), 0, Lk)
        idx_np = np.asarray(idx)
    counts = np.zeros((Lq, Lk), dtype=np.int8)
    np.add.at(counts, (np.arange(Lq)[:, None], idx_np), 1)
    return counts


def _sparsity_body(q_ref, k_ref, c_ref, out_ref):
    Lk = k_ref.shape[1]
    q = q_ref[0]
    k = k_ref[0]
    scores = lax.dot_general(q, k, (((1,), (1,)), ((), ())),
                             preferred_element_type=jnp.float32,
                             precision=lax.Precision.HIGHEST)
    c = c_ref[...].astype(jnp.float32)
    max_term = jnp.max(jnp.where(c > 0, scores, _NEG), axis=1)
    sum_term = jnp.sum(scores * c, axis=1) * (1.0 / Lk)
    out_ref[...] = (max_term - sum_term).reshape(out_ref.shape)


def _topk_attn_body(sp_ref, q_ref, k_ref, v_ref, out_ref, qtop_ref, mtop_ref,
                    *, n_top: int, scale: float):
    L, D = k_ref.shape[1], k_ref.shape[2]
    nt_pad = qtop_ref.shape[0]
    qtop_ref[...] = jnp.zeros((nt_pad, D), jnp.float32)
    iota = lax.broadcasted_iota(jnp.int32, (1, L), 1)

    def select(i, run):
        m = jnp.max(run)
        idx = jnp.min(jnp.where(run == m, iota, L))
        mtop_ref[i] = idx
        qtop_ref[pl.ds(i, 1), :] = q_ref[0, pl.ds(idx, 1), :]
        return jnp.where(iota == idx, _NEG, run)

    lax.fori_loop(0, n_top, select, sp_ref[0])

    k = k_ref[0]
    v = v_ref[0]
    qr = qtop_ref[...]
    st = lax.dot_general(qr, k, (((1,), (1,)), ((), ())),
                         preferred_element_type=jnp.float32,
                         precision=lax.Precision.HIGHEST) * scale
    mx = jnp.max(st, axis=1, keepdims=True)
    e = jnp.exp(st - mx)
    attn = e / jnp.sum(e, axis=1, keepdims=True)
    upd = lax.dot_general(attn, v, (((1,), (0,)), ((), ())),
                          preferred_element_type=jnp.float32,
                          precision=lax.Precision.HIGHEST)
    mean_v = jnp.sum(v, axis=0, keepdims=True) * (1.0 / L)
    out_ref[0] = jnp.broadcast_to(mean_v, (L, D))
    qtop_ref[...] = upd

    def scatter(i, carry):
        idx = mtop_ref[i]
        out_ref[0, pl.ds(idx, 1), :] = qtop_ref[pl.ds(i, 1), :]
        return carry

    lax.fori_loop(0, n_top, scatter, 0)


def kernel(queries, keys, values):
    B, L, H, D = queries.shape
    Lk = keys.shape[1]
    factor = 5
    sample_k = min(factor * math.ceil(math.log(Lk)), Lk)
    n_top = min(factor * math.ceil(math.log(L)), L)
    scale = 1.0 / math.sqrt(D)
    BH = B * H

    q = jnp.transpose(queries, (0, 2, 1, 3)).reshape(BH, L, D)
    k = jnp.transpose(keys, (0, 2, 1, 3)).reshape(BH, Lk, D)
    v = jnp.transpose(values, (0, 2, 1, 3)).reshape(BH, Lk, D)

    counts = jnp.asarray(_sample_count_matrix(L, Lk, sample_k))

    LT = min(512, L)
    n_lblk = L // LT
    sparsity = pl.pallas_call(
        _sparsity_body,
        grid=(n_lblk, BH),
        in_specs=[
            pl.BlockSpec((1, LT, D), lambda i, j: (j, i, 0)),
            pl.BlockSpec((1, Lk, D), lambda i, j: (j, 0, 0)),
            pl.BlockSpec((LT, Lk), lambda i, j: (i, 0)),
        ],
        out_specs=pl.BlockSpec((1, 1, LT), lambda i, j: (j, 0, i)),
        out_shape=jax.ShapeDtypeStruct((BH, 1, L), jnp.float32),
    )(q, k, counts)

    nt_pad = (n_top + 7) // 8 * 8
    ctx = pl.pallas_call(
        functools.partial(_topk_attn_body, n_top=n_top, scale=scale),
        grid=(BH,),
        in_specs=[
            pl.BlockSpec((1, 1, L), lambda j: (j, 0, 0)),
            pl.BlockSpec((1, L, D), lambda j: (j, 0, 0)),
            pl.BlockSpec((1, Lk, D), lambda j: (j, 0, 0)),
            pl.BlockSpec((1, Lk, D), lambda j: (j, 0, 0)),
        ],
        out_specs=pl.BlockSpec((1, L, D), lambda j: (j, 0, 0)),
        out_shape=jax.ShapeDtypeStruct((BH, L, D), jnp.float32),
        scratch_shapes=[
            pltpu.VMEM((nt_pad, D), jnp.float32),
            pltpu.SMEM((nt_pad,), jnp.int32),
        ],
    )(sparsity, q, k, v)

    return jnp.transpose(ctx.reshape(B, H, L, D), (0, 2, 1, 3))


# speedup vs baseline: 2.5743x; 1.3559x over previous
import functools
import math

import jax
import jax.numpy as jnp
import numpy as np
from jax import lax
from jax.experimental import pallas as pl
from jax.experimental.pallas import tpu as pltpu

_NEG = -3.0e38


@functools.lru_cache(maxsize=4)
def _sample_count_matrix(Lq: int, Lk: int, sample_k: int) -> np.ndarray:
    cpu = jax.devices("cpu")[0]
    with jax.ensure_compile_time_eval(), jax.default_device(cpu):
        idx = jax.random.randint(jax.random.key(42), (Lq, sample_k), 0, Lk)
        idx_np = np.asarray(idx)
    counts = np.zeros((Lq, Lk), dtype=np.int8)
    np.add.at(counts, (np.arange(Lq)[:, None], idx_np), 1)
    return counts


def _dot_bf16x3(a, b):
    dims = (((1,), (1,)), ((), ()))
    ah = a.astype(jnp.bfloat16)
    al = (a - ah.astype(jnp.float32)).astype(jnp.bfloat16)
    bh = b.astype(jnp.bfloat16)
    bl = (b - bh.astype(jnp.float32)).astype(jnp.bfloat16)
    f32 = jnp.float32
    return (lax.dot_general(ah, bl, dims, preferred_element_type=f32)
            + lax.dot_general(al, bh, dims, preferred_element_type=f32)
            + lax.dot_general(ah, bh, dims, preferred_element_type=f32))


def _sparsity_body(q_ref, k_ref, c_ref, out_ref):
    Lk = k_ref.shape[1]
    q = q_ref[0]
    k = k_ref[0]
    scores = _dot_bf16x3(q, k)
    c = c_ref[...].astype(jnp.float32)
    max_term = jnp.max(jnp.where(c > 0, scores, _NEG), axis=1)
    sum_term = jnp.sum(scores * c, axis=1) * (1.0 / Lk)
    out_ref[...] = (max_term - sum_term).reshape(out_ref.shape)


def _topk_attn_body(sp_ref, q_ref, k_ref, v_ref, out_ref, qtop_ref, mtop_ref,
                    *, n_top: int, scale: float):
    L, D = k_ref.shape[1], k_ref.shape[2]
    nt_pad = qtop_ref.shape[0]
    qtop_ref[...] = jnp.zeros((nt_pad, D), jnp.float32)
    iota = lax.broadcasted_iota(jnp.int32, (1, L), 1)

    def select(i, run):
        m = jnp.max(run)
        idx = jnp.min(jnp.where(run == m, iota, L))
        mtop_ref[i] = idx
        qtop_ref[pl.ds(i, 1), :] = q_ref[0, pl.ds(idx, 1), :]
        return jnp.where(iota == idx, _NEG, run)

    lax.fori_loop(0, n_top, select, sp_ref[0])

    k = k_ref[0]
    v = v_ref[0]
    qr = qtop_ref[...]
    st = _dot_bf16x3(qr, k) * scale
    mx = jnp.max(st, axis=1, keepdims=True)
    e = jnp.exp(st - mx)
    attn = e / jnp.sum(e, axis=1, keepdims=True)
    upd = lax.dot_general(attn, v, (((1,), (0,)), ((), ())),
                          preferred_element_type=jnp.float32,
                          precision=lax.Precision.HIGHEST)
    mean_v = jnp.sum(v, axis=0, keepdims=True) * (1.0 / L)
    out_ref[0] = jnp.broadcast_to(mean_v, (L, D))
    qtop_ref[...] = upd

    def scatter(i, carry):
        idx = mtop_ref[i]
        out_ref[0, pl.ds(idx, 1), :] = qtop_ref[pl.ds(i, 1), :]
        return carry

    lax.fori_loop(0, n_top, scatter, 0)


def kernel(queries, keys, values):
    B, L, H, D = queries.shape
    Lk = keys.shape[1]
    factor = 5
    sample_k = min(factor * math.ceil(math.log(Lk)), Lk)
    n_top = min(factor * math.ceil(math.log(L)), L)
    scale = 1.0 / math.sqrt(D)
    BH = B * H

    q = jnp.transpose(queries, (0, 2, 1, 3)).reshape(BH, L, D)
    k = jnp.transpose(keys, (0, 2, 1, 3)).reshape(BH, Lk, D)
    v = jnp.transpose(values, (0, 2, 1, 3)).reshape(BH, Lk, D)

    counts = jnp.asarray(_sample_count_matrix(L, Lk, sample_k))

    LT = min(512, L)
    n_lblk = L // LT
    sparsity = pl.pallas_call(
        _sparsity_body,
        grid=(n_lblk, BH),
        in_specs=[
            pl.BlockSpec((1, LT, D), lambda i, j: (j, i, 0)),
            pl.BlockSpec((1, Lk, D), lambda i, j: (j, 0, 0)),
            pl.BlockSpec((LT, Lk), lambda i, j: (i, 0)),
        ],
        out_specs=pl.BlockSpec((1, 1, LT), lambda i, j: (j, 0, i)),
        out_shape=jax.ShapeDtypeStruct((BH, 1, L), jnp.float32),
    )(q, k, counts)

    nt_pad = (n_top + 7) // 8 * 8
    ctx = pl.pallas_call(
        functools.partial(_topk_attn_body, n_top=n_top, scale=scale),
        grid=(BH,),
        in_specs=[
            pl.BlockSpec((1, 1, L), lambda j: (j, 0, 0)),
            pl.BlockSpec((1, L, D), lambda j: (j, 0, 0)),
            pl.BlockSpec((1, Lk, D), lambda j: (j, 0, 0)),
            pl.BlockSpec((1, Lk, D), lambda j: (j, 0, 0)),
        ],
        out_specs=pl.BlockSpec((1, L, D), lambda j: (j, 0, 0)),
        out_shape=jax.ShapeDtypeStruct((BH, L, D), jnp.float32),
        scratch_shapes=[
            pltpu.VMEM((nt_pad, D), jnp.float32),
            pltpu.SMEM((nt_pad,), jnp.int32),
        ],
    )(sparsity, q, k, v)

    return jnp.transpose(ctx.reshape(B, H, L, D), (0, 2, 1, 3))


# speedup vs baseline: 3.4328x; 1.3335x over previous
import functools
import math

import jax
import jax.numpy as jnp
import numpy as np
from jax import lax
from jax.experimental import pallas as pl
from jax.experimental.pallas import tpu as pltpu

_NEG = -3.0e38


@functools.lru_cache(maxsize=4)
def _sample_count_matrix(Lq: int, Lk: int, sample_k: int) -> np.ndarray:
    cpu = jax.devices("cpu")[0]
    with jax.ensure_compile_time_eval(), jax.default_device(cpu):
        idx = jax.random.randint(jax.random.key(42), (Lq, sample_k), 0, Lk)
        idx_np = np.asarray(idx)
    counts = np.zeros((Lq, Lk), dtype=np.int8)
    np.add.at(counts, (np.arange(Lq)[:, None], idx_np), 1)
    return counts


def _dot_bf16x3(a, b):
    dims = (((1,), (1,)), ((), ()))
    ah = a.astype(jnp.bfloat16)
    al = (a - ah.astype(jnp.float32)).astype(jnp.bfloat16)
    bh = b.astype(jnp.bfloat16)
    bl = (b - bh.astype(jnp.float32)).astype(jnp.bfloat16)
    f32 = jnp.float32
    return (lax.dot_general(ah, bl, dims, preferred_element_type=f32)
            + lax.dot_general(al, bh, dims, preferred_element_type=f32)
            + lax.dot_general(ah, bh, dims, preferred_element_type=f32))


def _sparsity_body(q_ref, k_ref, c_ref, out_ref):
    Lk = k_ref.shape[1]
    q = q_ref[0]
    k = k_ref[0]
    scores = _dot_bf16x3(q, k)
    c = c_ref[...].astype(jnp.float32)
    max_term = jnp.max(jnp.where(c > 0, scores, _NEG), axis=1)
    sum_term = jnp.sum(scores * c, axis=1) * (1.0 / Lk)
    out_ref[...] = (max_term - sum_term).reshape(out_ref.shape)


def _topk_body(sp_ref, out_ref, *, n_top: int):
    BH, L = sp_ref.shape
    out_ref[...] = jnp.zeros(out_ref.shape, jnp.int32)
    iota = lax.broadcasted_iota(jnp.int32, (BH, L), 1)
    run = sp_ref[...]
    for i in range(n_top):
        m = jnp.max(run, axis=1, keepdims=True)
        idx = jnp.min(jnp.where(run == m, iota, L), axis=1, keepdims=True)
        out_ref[:, i:i + 1] = idx
        run = jnp.where(iota == idx, _NEG, run)


def _topk_attn_body(mt_ref, sp_ref, q_ref, k_ref, v_ref, out_ref, qtop_ref,
                    *, n_top: int, scale: float):
    L, D = k_ref.shape[1], k_ref.shape[2]
    nt_pad = qtop_ref.shape[0]
    qtop_ref[...] = jnp.zeros((nt_pad, D), jnp.float32)

    def select(i, carry):
        idx = mt_ref[0, 0, i]
        qtop_ref[pl.ds(i, 1), :] = q_ref[0, pl.ds(idx, 1), :]
        return carry

    lax.fori_loop(0, n_top, select, 0)

    k = k_ref[0]
    v = v_ref[0]
    qr = qtop_ref[...]
    st = _dot_bf16x3(qr, k) * scale
    mx = jnp.max(st, axis=1, keepdims=True)
    e = jnp.exp(st - mx)
    attn = e / jnp.sum(e, axis=1, keepdims=True)
    upd = lax.dot_general(attn, v, (((1,), (0,)), ((), ())),
                          preferred_element_type=jnp.float32,
                          precision=lax.Precision.HIGHEST)
    mean_v = jnp.sum(v, axis=0, keepdims=True) * (1.0 / L)
    out_ref[0] = jnp.broadcast_to(mean_v, (L, D))
    qtop_ref[...] = upd

    def scatter(i, carry):
        idx = mt_ref[0, 0, i]
        out_ref[0, pl.ds(idx, 1), :] = qtop_ref[pl.ds(i, 1), :]
        return carry

    lax.fori_loop(0, n_top, scatter, 0)


def kernel(queries, keys, values):
    B, L, H, D = queries.shape
    Lk = keys.shape[1]
    factor = 5
    sample_k = min(factor * math.ceil(math.log(Lk)), Lk)
    n_top = min(factor * math.ceil(math.log(L)), L)
    scale = 1.0 / math.sqrt(D)
    BH = B * H

    q = jnp.transpose(queries, (0, 2, 1, 3)).reshape(BH, L, D)
    k = jnp.transpose(keys, (0, 2, 1, 3)).reshape(BH, Lk, D)
    v = jnp.transpose(values, (0, 2, 1, 3)).reshape(BH, Lk, D)

    counts = jnp.asarray(_sample_count_matrix(L, Lk, sample_k))

    LT = min(512, L)
    n_lblk = L // LT
    sparsity = pl.pallas_call(
        _sparsity_body,
        grid=(n_lblk, BH),
        in_specs=[
            pl.BlockSpec((1, LT, D), lambda i, j: (j, i, 0)),
            pl.BlockSpec((1, Lk, D), lambda i, j: (j, 0, 0)),
            pl.BlockSpec((LT, Lk), lambda i, j: (i, 0)),
        ],
        out_specs=pl.BlockSpec((1, 1, LT), lambda i, j: (j, 0, i)),
        out_shape=jax.ShapeDtypeStruct((BH, 1, L), jnp.float32),
    )(q, k, counts)

    nt_pad = (n_top + 7) // 8 * 8
    mtop = pl.pallas_call(
        functools.partial(_topk_body, n_top=n_top),
        grid=(1,),
        in_specs=[pl.BlockSpec((BH, L), lambda j: (0, 0))],
        out_specs=pl.BlockSpec((BH, nt_pad), lambda j: (0, 0)),
        out_shape=jax.ShapeDtypeStruct((BH, nt_pad), jnp.int32),
    )(sparsity.reshape(BH, L))

    ctx = pl.pallas_call(
        functools.partial(_topk_attn_body, n_top=n_top, scale=scale),
        grid=(BH,),
        in_specs=[
            pl.BlockSpec((1, 1, nt_pad), lambda j: (j, 0, 0),
                         memory_space=pltpu.SMEM),
            pl.BlockSpec((1, 1, L), lambda j: (j, 0, 0)),
            pl.BlockSpec((1, L, D), lambda j: (j, 0, 0)),
            pl.BlockSpec((1, Lk, D), lambda j: (j, 0, 0)),
            pl.BlockSpec((1, Lk, D), lambda j: (j, 0, 0)),
        ],
        out_specs=pl.BlockSpec((1, L, D), lambda j: (j, 0, 0)),
        out_shape=jax.ShapeDtypeStruct((BH, L, D), jnp.float32),
        scratch_shapes=[
            pltpu.VMEM((nt_pad, D), jnp.float32),
        ],
    )(mtop.reshape(BH, 1, nt_pad), sparsity, q, k, v)

    return jnp.transpose(ctx.reshape(B, H, L, D), (0, 2, 1, 3))


# speedup vs baseline: 3.6311x; 1.0578x over previous
import functools
import math

import jax
import jax.numpy as jnp
import numpy as np
from jax import lax
from jax.experimental import pallas as pl
from jax.experimental.pallas import tpu as pltpu

_NEG = -3.0e38


@functools.lru_cache(maxsize=4)
def _sample_count_matrix(Lq: int, Lk: int, sample_k: int) -> np.ndarray:
    cpu = jax.devices("cpu")[0]
    with jax.ensure_compile_time_eval(), jax.default_device(cpu):
        idx = jax.random.randint(jax.random.key(42), (Lq, sample_k), 0, Lk)
        idx_np = np.asarray(idx)
    counts = np.zeros((Lq, Lk), dtype=np.int8)
    np.add.at(counts, (np.arange(Lq)[:, None], idx_np), 1)
    cf = counts.astype(np.float32)
    maskadd = np.where(counts > 0, 0.0, _NEG).astype(np.float32)
    return cf, maskadd


def _dot_bf16x3(a, b):
    dims = (((1,), (1,)), ((), ()))
    ah = a.astype(jnp.bfloat16)
    al = (a - ah.astype(jnp.float32)).astype(jnp.bfloat16)
    bh = b.astype(jnp.bfloat16)
    bl = (b - bh.astype(jnp.float32)).astype(jnp.bfloat16)
    f32 = jnp.float32
    return (lax.dot_general(ah, bl, dims, preferred_element_type=f32)
            + lax.dot_general(al, bh, dims, preferred_element_type=f32)
            + lax.dot_general(ah, bh, dims, preferred_element_type=f32))


def _sparsity_body(q_ref, k_ref, cf_ref, ma_ref, out_ref):
    Lk = k_ref.shape[1]
    q = q_ref[0]
    k = k_ref[0]
    scores = _dot_bf16x3(q, k)
    max_term = jnp.max(scores + ma_ref[...], axis=1)
    sum_term = jnp.sum(scores * cf_ref[...], axis=1) * (1.0 / Lk)
    out_ref[...] = (max_term - sum_term).reshape(out_ref.shape)


def _topk_body(sp_ref, out_ref, *, n_top: int):
    BH, L = sp_ref.shape
    out_ref[...] = jnp.zeros(out_ref.shape, jnp.int32)
    iota = lax.broadcasted_iota(jnp.int32, (BH, L), 1)
    run = sp_ref[...]
    for i in range(n_top):
        m = jnp.max(run, axis=1, keepdims=True)
        idx = jnp.min(jnp.where(run == m, iota, L), axis=1, keepdims=True)
        out_ref[:, i:i + 1] = idx
        run = jnp.where(iota == idx, _NEG, run)


def _topk_attn_body(mt_ref, sp_ref, q_ref, k_ref, v_ref, out_ref, qtop_ref,
                    *, n_top: int, scale: float):
    L, D = k_ref.shape[1], k_ref.shape[2]
    nt_pad = qtop_ref.shape[0]
    qtop_ref[...] = jnp.zeros((nt_pad, D), jnp.float32)

    def select(i, carry):
        idx = mt_ref[0, 0, i]
        qtop_ref[pl.ds(i, 1), :] = q_ref[0, pl.ds(idx, 1), :]
        return carry

    lax.fori_loop(0, n_top, select, 0)

    k = k_ref[0]
    v = v_ref[0]
    qr = qtop_ref[...]
    st = _dot_bf16x3(qr, k) * scale
    mx = jnp.max(st, axis=1, keepdims=True)
    e = jnp.exp(st - mx)
    attn = e / jnp.sum(e, axis=1, keepdims=True)
    upd = lax.dot_general(attn, v, (((1,), (0,)), ((), ())),
                          preferred_element_type=jnp.float32,
                          precision=lax.Precision.HIGHEST)
    mean_v = jnp.sum(v, axis=0, keepdims=True) * (1.0 / L)
    out_ref[0] = jnp.broadcast_to(mean_v, (L, D))
    qtop_ref[...] = upd

    def scatter(i, carry):
        idx = mt_ref[0, 0, i]
        out_ref[0, pl.ds(idx, 1), :] = qtop_ref[pl.ds(i, 1), :]
        return carry

    lax.fori_loop(0, n_top, scatter, 0)


def kernel(queries, keys, values):
    B, L, H, D = queries.shape
    Lk = keys.shape[1]
    factor = 5
    sample_k = min(factor * math.ceil(math.log(Lk)), Lk)
    n_top = min(factor * math.ceil(math.log(L)), L)
    scale = 1.0 / math.sqrt(D)
    BH = B * H

    q = jnp.transpose(queries, (0, 2, 1, 3)).reshape(BH, L, D)
    k = jnp.transpose(keys, (0, 2, 1, 3)).reshape(BH, Lk, D)
    v = jnp.transpose(values, (0, 2, 1, 3)).reshape(BH, Lk, D)

    cf_np, ma_np = _sample_count_matrix(L, Lk, sample_k)
    cf = jnp.asarray(cf_np)
    ma = jnp.asarray(ma_np)

    LT = min(512, L)
    n_lblk = L // LT
    sparsity = pl.pallas_call(
        _sparsity_body,
        grid=(n_lblk, BH),
        in_specs=[
            pl.BlockSpec((1, LT, D), lambda i, j: (j, i, 0)),
            pl.BlockSpec((1, Lk, D), lambda i, j: (j, 0, 0)),
            pl.BlockSpec((LT, Lk), lambda i, j: (i, 0)),
            pl.BlockSpec((LT, Lk), lambda i, j: (i, 0)),
        ],
        out_specs=pl.BlockSpec((1, 1, LT), lambda i, j: (j, 0, i)),
        out_shape=jax.ShapeDtypeStruct((BH, 1, L), jnp.float32),
    )(q, k, cf, ma)

    nt_pad = (n_top + 7) // 8 * 8
    mtop = pl.pallas_call(
        functools.partial(_topk_body, n_top=n_top),
        grid=(1,),
        in_specs=[pl.BlockSpec((BH, L), lambda j: (0, 0))],
        out_specs=pl.BlockSpec((BH, nt_pad), lambda j: (0, 0)),
        out_shape=jax.ShapeDtypeStruct((BH, nt_pad), jnp.int32),
    )(sparsity.reshape(BH, L))

    ctx = pl.pallas_call(
        functools.partial(_topk_attn_body, n_top=n_top, scale=scale),
        grid=(BH,),
        in_specs=[
            pl.BlockSpec((1, 1, nt_pad), lambda j: (j, 0, 0),
                         memory_space=pltpu.SMEM),
            pl.BlockSpec((1, 1, L), lambda j: (j, 0, 0)),
            pl.BlockSpec((1, L, D), lambda j: (j, 0, 0)),
            pl.BlockSpec((1, Lk, D), lambda j: (j, 0, 0)),
            pl.BlockSpec((1, Lk, D), lambda j: (j, 0, 0)),
        ],
        out_specs=pl.BlockSpec((1, L, D), lambda j: (j, 0, 0)),
        out_shape=jax.ShapeDtypeStruct((BH, L, D), jnp.float32),
        scratch_shapes=[
            pltpu.VMEM((nt_pad, D), jnp.float32),
        ],
    )(mtop.reshape(BH, 1, nt_pad), sparsity, q, k, v)

    return jnp.transpose(ctx.reshape(B, H, L, D), (0, 2, 1, 3))
